# baseline (speedup 1.0000x reference)
import functools

import jax
import jax.numpy as jnp
from jax import lax
from jax.experimental import pallas as pl
from jax.experimental.pallas import tpu as pltpu
from jax.experimental.pallas import tpu_sc as plsc

B, C, N_IN = 4, 64, 8192
K, P = 4, 4
N_OUT = 32768
ROWS = B * C
NW = 32
ROWS_PER_W = ROWS // NW
NKROWS = N_IN * K
NPL = 2 * P
L = 16
Q = 2048
NQ = N_IN // Q


def _unpool_body(x_hbm, idxq_hbm, planes_hbm, out_hbm,
                 acc, xr, irq, gpl, tables_sp, semA, semB):
    nc = 2
    wid = lax.axis_index("s") * nc + lax.axis_index("c")
    lane = jnp.arange(L, dtype=jnp.int32)
    zero = jnp.zeros((L,), jnp.float32)
    sems = [semA, semB]

    @pl.when(lax.axis_index("s") == 0)
    def _():
        pltpu.sync_copy(planes_hbm, tables_sp)

    plsc.subcore_barrier()

    def compute_quarter(q, buf):
        def inner(t, _):
            n0 = t * L
            xv = xr[pl.ds(q * Q + n0, L)]
            for p in range(P):
                smv = gpl[buf, p, pl.ds(n0, L)]
                iwv = plsc.bitcast(gpl[buf, P + p, pl.ds(n0, L)], jnp.float32)
                plsc.addupdate_scatter(acc, [smv], xv * iwv)
            return 0

        lax.fori_loop(0, Q // L, inner, 0)

    def row_body(i, _):
        r = wid * ROWS_PER_W + i
        pltpu.sync_copy(x_hbm.at[r], xr)

        copies = [None, None]
        for q in range(NQ):
            buf = q % 2
            pltpu.sync_copy(idxq_hbm.at[r * NQ + q], irq.at[buf])

            def gix_body(t, _):
                n0 = t * L
                kv = irq[buf, pl.ds(n0, L)]
                irq[buf, pl.ds(n0, L)] = (lane + (q * Q + n0)) * K + kv
                return 0

            lax.fori_loop(0, Q // L, gix_body, 0)
            cps = []
            for w in range(NPL):
                cp = pltpu.make_async_copy(
                    tables_sp.at[w].at[irq.at[buf]], gpl.at[buf, w], sems[buf])
                cp.start()
                cps.append(cp)
            copies[buf] = cps
            if q == 0:
                def zbody(j, _):
                    base = j * (L * 8)
                    for u in range(8):
                        acc[pl.ds(base + u * L, L)] = zero
                    return 0

                lax.fori_loop(0, N_OUT // (L * 8), zbody, 0)
            else:
                for cp in copies[1 - buf]:
                    cp.wait()
                compute_quarter(q - 1, 1 - buf)
        for cp in copies[(NQ - 1) % 2]:
            cp.wait()
        compute_quarter(NQ - 1, (NQ - 1) % 2)
        pltpu.sync_copy(acc, out_hbm.at[r])
        return 0

    lax.fori_loop(0, ROWS_PER_W, row_body, 0)


@jax.jit
def _unpool(xf, idxq, planes):
    mesh = plsc.VectorSubcoreMesh(core_axis_name="c", subcore_axis_name="s")
    f = functools.partial(
        pl.kernel,
        mesh=mesh,
        compiler_params=pltpu.CompilerParams(
            needs_layout_passes=False, use_tc_tiling_on_sc=False),
        out_type=jax.ShapeDtypeStruct((ROWS, N_OUT), jnp.float32),
        scratch_types=[
            pltpu.VMEM((N_OUT,), jnp.float32),
            pltpu.VMEM((N_IN,), jnp.float32),
            pltpu.VMEM((2, Q), jnp.int32),
            pltpu.VMEM((2, NPL, Q), jnp.int32),
            pltpu.VMEM_SHARED((NPL, NKROWS), jnp.int32),
            pltpu.SemaphoreType.DMA,
            pltpu.SemaphoreType.DMA,
        ],
    )(_unpool_body)
    return f(xf, idxq, planes)


def kernel(x, idx_mask, sample_map, interp_weights):
    xf = x.reshape(ROWS, N_IN)
    idxq = idx_mask.reshape(ROWS * NQ, Q).astype(jnp.int32)
    smT = sample_map.reshape(NKROWS, P).astype(jnp.int32).T
    iwT = lax.bitcast_convert_type(
        interp_weights.reshape(NKROWS, P), jnp.int32).T
    planes = jnp.concatenate([smT, iwT], axis=0)
    out = _unpool(xf, idxq, planes)
    return out.reshape(B, C, N_OUT)
